# baseline (speedup 1.0000x reference)
import functools

import jax
import jax.numpy as jnp
from jax import lax
from jax.experimental import pallas as pl
from jax.experimental.pallas import tpu as pltpu
from jax.experimental.pallas import tpu_sc as plsc

N = 10000
NPAD = 10240
D = 128
E = 320000
EALL = E + N
EPAD = 330240
NS = 16
EPT = EPAD // NS
EPT4 = EPT * 4
NEC = 4
ECH = EPT4 // NEC
NPS = NPAD // NS
SLW = NPS * 4
NPAD4 = NPAD * 4
XCH = 80
NXC = NPS // XCH

_f32 = jnp.float32
_i32 = jnp.int32


def _rsqrt_newton(d):
    d = jnp.maximum(d, 0.25)
    i = lax.bitcast_convert_type(d, _i32)
    i = 0x5F3759DF - lax.shift_right_logical(i, 1)
    y = lax.bitcast_convert_type(i, _f32)
    for _ in range(3):
        y = y * (1.5 - 0.5 * d * y * y)
    return y


def _tanh16(v):
    vm = jnp.minimum(jnp.maximum(v, -15.0), 15.0)
    e = jnp.exp(2.0 * vm)
    return 1.0 - 2.0 / (e + 1.0)


_sc_mesh = plsc.VectorSubcoreMesh(
    core_axis_name="c", subcore_axis_name="s", num_cores=1
)


@functools.partial(
    pl.kernel,
    out_type=(
        jax.ShapeDtypeStruct((NPAD4,), _f32),
        jax.ShapeDtypeStruct((NPAD4,), _f32),
    ),
    mesh=_sc_mesh,
    compiler_params=pltpu.CompilerParams(
        needs_layout_passes=False, use_tc_tiling_on_sc=False
    ),
    scratch_types=[
        pltpu.VMEM_SHARED((NPAD,), _f32),
        pltpu.VMEM_SHARED((NPAD4,), _f32),
        pltpu.VMEM_SHARED((NPAD4,), _f32),
        pltpu.VMEM((ECH,), _i32),
        pltpu.VMEM((ECH,), _i32),
        pltpu.VMEM((ECH,), _f32),
        pltpu.VMEM((NPAD,), _f32),
        pltpu.VMEM((SLW,), _f32),
        pltpu.VMEM((SLW,), _f32),
        pltpu.VMEM((SLW,), _f32),
        pltpu.VMEM((XCH, D), _f32),
        pltpu.VMEM((512 * 16,), _f32),
        pltpu.VMEM((12, 16), _f32),
        pltpu.VMEM((4, 16), _f32),
        pltpu.VMEM((16,), _f32),
    ],
)
def _sc_gcn(src4_hbm, dst4_hbm, dst_hbm, x_hbm, w1_hbm, wp_hbm, bp_hbm,
            out_hbm, h3_hbm,
            deg_sh, gs_sh, acc_sh, ia_v, ib_v, up_v, dinv_v,
            zf_v, sl_v, pa_v, xc_v, w1_v, wp_v, bp_v, tb_v):
    sid = lax.axis_index("s")
    iota = lax.iota(_i32, 16)
    nodeoff = lax.shift_right_logical(iota, 2)
    blk = jnp.bitwise_and(iota, 12)
    ones16 = jnp.full((16,), 1.0, _f32)
    zeros16 = jnp.zeros((16,), _f32)
    nbase = sid * NPS

    pltpu.sync_copy(w1_hbm, w1_v)
    pltpu.sync_copy(wp_hbm, wp_v)
    pltpu.sync_copy(bp_hbm, bp_v)

    def zfill(j, c):
        zf_v[pl.ds(j * 16, 16)] = zeros16
        return c

    lax.fori_loop(0, SLW // 16, zfill, 0)

    def ofill(j, c):
        up_v[pl.ds(j * 16, 16)] = ones16
        return c

    lax.fori_loop(0, ECH // 16, ofill, 0)

    pltpu.sync_copy(zf_v.at[pl.ds(0, NPS)], deg_sh.at[pl.ds(nbase, NPS)])
    pltpu.sync_copy(zf_v, acc_sh.at[pl.ds(nbase * 4, SLW)])
    plsc.subcore_barrier()

    pltpu.sync_copy(dst_hbm.at[pl.ds(sid * EPT, EPT)], ia_v)
    pltpu.sync_copy(up_v, deg_sh.at[ia_v], add=True)
    plsc.subcore_barrier()

    pltpu.sync_copy(deg_sh, dinv_v)

    def rbody(j, c):
        dinv_v[pl.ds(j * 16, 16)] = _rsqrt_newton(dinv_v[pl.ds(j * 16, 16)])
        return c

    lax.fori_loop(0, NPAD // 16, rbody, 0)

    def xchunk(cc, k):
        start = nbase + cc * XCH

        @pl.when(start < N)
        def _():
            pltpu.sync_copy(x_hbm.at[pl.ds(start, XCH), :], xc_v)

            def group(g, kk):
                rows16 = g * 16 + iota

                def kbody(kd, accs):
                    a0, a1, a2, a3 = accs
                    kcol = jnp.broadcast_to(kd, (16,))
                    xv = plsc.load_gather(xc_v, [rows16, kcol])
                    wb = jnp.broadcast_to(kd * 64, (16,)) + iota
                    a0 = a0 + xv * plsc.load_gather(w1_v, [wb])
                    a1 = a1 + xv * plsc.load_gather(w1_v, [wb + 16])
                    a2 = a2 + xv * plsc.load_gather(w1_v, [wb + 32])
                    a3 = a3 + xv * plsc.load_gather(w1_v, [wb + 48])
                    return (a0, a1, a2, a3)

                accs = lax.fori_loop(
                    0, D, kbody, (zeros16, zeros16, zeros16, zeros16)
                )
                dix = plsc.load_gather(dinv_v, [start + rows16])
                lidx = (cc * XCH + rows16) * 4
                plsc.store_scatter(sl_v, [lidx], accs[0] * dix)
                plsc.store_scatter(sl_v, [lidx + 1], accs[1] * dix)
                plsc.store_scatter(sl_v, [lidx + 2], accs[2] * dix)
                plsc.store_scatter(sl_v, [lidx + 3], accs[3] * dix)
                return kk

            lax.fori_loop(0, XCH // 16, group, 0)

        @pl.when(start >= N)
        def _():
            def zg2(j, kk):
                sl_v[pl.ds(cc * XCH * 4 + j * 16, 16)] = zeros16
                return kk

            lax.fori_loop(0, XCH * 4 // 16, zg2, 0)

        return k

    lax.fori_loop(0, NXC, xchunk, 0)
    pltpu.sync_copy(sl_v, gs_sh.at[pl.ds(nbase * 4, SLW)])
    plsc.subcore_barrier()

    def edge_phase():
        def body(c, k):
            base = sid * EPT4 + c * ECH
            pltpu.sync_copy(src4_hbm.at[pl.ds(base, ECH)], ia_v)
            pltpu.sync_copy(dst4_hbm.at[pl.ds(base, ECH)], ib_v)
            pltpu.sync_copy(gs_sh.at[ia_v], up_v)
            pltpu.sync_copy(up_v, acc_sh.at[ib_v], add=True)
            return k

        lax.fori_loop(0, NEC, body, 0)
        plsc.subcore_barrier()

    def epilogue(layer):
        pltpu.sync_copy(acc_sh.at[pl.ds(nbase * 4, SLW)], pa_v)
        bvec = bp_v[layer, :]
        w0 = wp_v[layer * 4 + 0, :]
        w1r = wp_v[layer * 4 + 1, :]
        w2r = wp_v[layer * 4 + 2, :]
        w3r = wp_v[layer * 4 + 3, :]

        def ebody(j, c):
            v = pa_v[pl.ds(j * 16, 16)]
            dix = plsc.load_gather(dinv_v, [nbase + j * 4 + nodeoff])
            v = v * dix + bvec
            t = _tanh16(v)
            tb_v[...] = t
            acc = plsc.load_gather(tb_v, [blk]) * w0
            acc = acc + plsc.load_gather(tb_v, [blk + 1]) * w1r
            acc = acc + plsc.load_gather(tb_v, [blk + 2]) * w2r
            acc = acc + plsc.load_gather(tb_v, [blk + 3]) * w3r
            sl_v[pl.ds(j * 16, 16)] = acc * dix
            return c

        lax.fori_loop(0, SLW // 16, ebody, 0)
        pltpu.sync_copy(zf_v, acc_sh.at[pl.ds(nbase * 4, SLW)])
        pltpu.sync_copy(sl_v, gs_sh.at[pl.ds(nbase * 4, SLW)])
        plsc.subcore_barrier()

    edge_phase()
    epilogue(0)
    edge_phase()
    epilogue(1)
    edge_phase()

    pltpu.sync_copy(acc_sh.at[pl.ds(nbase * 4, SLW)], pa_v)
    b3vec = bp_v[2, :]
    blvec = bp_v[3, :]
    wl0 = wp_v[8, :]
    wl1 = wp_v[9, :]
    wl2 = wp_v[10, :]
    wl3 = wp_v[11, :]

    def fbody(j, c):
        v = pa_v[pl.ds(j * 16, 16)]
        dix = plsc.load_gather(dinv_v, [nbase + j * 4 + nodeoff])
        h3 = v * dix + b3vec
        tb_v[...] = h3
        acc = plsc.load_gather(tb_v, [blk]) * wl0
        acc = acc + plsc.load_gather(tb_v, [blk + 1]) * wl1
        acc = acc + plsc.load_gather(tb_v, [blk + 2]) * wl2
        acc = acc + plsc.load_gather(tb_v, [blk + 3]) * wl3
        sl_v[pl.ds(j * 16, 16)] = h3
        pa_v[pl.ds(j * 16, 16)] = acc + blvec
        return c

    lax.fori_loop(0, SLW // 16, fbody, 0)
    pltpu.sync_copy(sl_v, h3_hbm.at[pl.ds(nbase * 4, SLW)])
    pltpu.sync_copy(pa_v, out_hbm.at[pl.ds(nbase * 4, SLW)])


def kernel(x, edge_index, W1, b1, W2, b2, W3, b3, Wl, bl):
    src = edge_index[0].astype(_i32)
    dst = edge_index[1].astype(_i32)
    loop = jnp.arange(N, dtype=_i32)
    padn = N + jnp.arange(EPAD - EALL, dtype=_i32) % (NPAD - N)
    src_all = jnp.concatenate([src, loop, padn])
    dst_all = jnp.concatenate([dst, loop, padn])
    four = jnp.arange(4, dtype=_i32)
    src4 = (src_all[:, None] * 4 + four[None, :]).reshape(-1)
    dst4 = (dst_all[:, None] * 4 + four[None, :]).reshape(-1)

    w1f = jnp.tile(W1.reshape(512, 1), (1, 16)).reshape(-1)
    w3p = jnp.pad(W3, ((0, 0), (0, 2)))
    wlp = jnp.pad(Wl, ((0, 2), (0, 0)))
    wp = jnp.concatenate(
        [jnp.tile(W2, (1, 4)), jnp.tile(w3p, (1, 4)), jnp.tile(wlp, (1, 4))]
    )
    bp = jnp.stack(
        [jnp.tile(b1, 4), jnp.tile(b2, 4),
         jnp.tile(jnp.pad(b3, (0, 2)), 4), jnp.tile(bl, 4)]
    )

    out_full, h3_full = _sc_gcn(src4, dst4, dst_all, x, w1f, wp, bp)
    return (out_full.reshape(NPAD, 4)[:N],
            h3_full.reshape(NPAD, 4)[:N, :2])


# speedup vs baseline: 2.6021x; 2.6021x over previous
import functools

import jax
import jax.numpy as jnp
from jax import lax
from jax.experimental import pallas as pl
from jax.experimental.pallas import tpu as pltpu
from jax.experimental.pallas import tpu_sc as plsc

N = 10000
NPAD = 10240
D = 128
E = 320000
NS = 16
EPT = E // NS
ECH = 4000
NCH = EPT // ECH
ECH4 = ECH * 4
NPS = NPAD // NS
SLW = NPS * 4
NPAD4 = NPAD * 4
NPAD2 = NPAD * 2
XCH = 80
NXC = NPS // XCH

_f32 = jnp.float32
_i32 = jnp.int32


def _rsqrt_newton(d):
    d = jnp.maximum(d, 0.25)
    i = lax.bitcast_convert_type(d, _i32)
    i = 0x5F3759DF - lax.shift_right_logical(i, 1)
    y = lax.bitcast_convert_type(i, _f32)
    for _ in range(3):
        y = y * (1.5 - 0.5 * d * y * y)
    return y


def _bf16r(x):
    i = lax.bitcast_convert_type(x, _i32)
    r = i + 0x7FFF + jnp.bitwise_and(lax.shift_right_logical(i, 16), 1)
    r = jnp.bitwise_and(r, -65536)
    return lax.bitcast_convert_type(r, _f32)


def _tanh16(v):
    vm = jnp.minimum(jnp.maximum(v, -15.0), 15.0)
    e = jnp.exp(2.0 * vm)
    return 1.0 - 2.0 / (e + 1.0)


_sc_mesh = plsc.VectorSubcoreMesh(
    core_axis_name="c", subcore_axis_name="s", num_cores=1
)


@functools.partial(
    pl.kernel,
    out_type=(
        jax.ShapeDtypeStruct((NPAD4,), _f32),
        jax.ShapeDtypeStruct((NPAD2,), _f32),
    ),
    mesh=_sc_mesh,
    compiler_params=pltpu.CompilerParams(
        needs_layout_passes=False, use_tc_tiling_on_sc=False
    ),
    scratch_types=[
        pltpu.VMEM_SHARED((NPAD,), _f32),
        pltpu.VMEM_SHARED((NPAD4,), _f32),
        pltpu.VMEM_SHARED((NPAD4,), _f32),
        pltpu.VMEM((ECH,), _i32),
        pltpu.VMEM((ECH,), _i32),
        pltpu.VMEM((ECH4,), _i32),
        pltpu.VMEM((ECH4,), _i32),
        pltpu.VMEM((ECH4,), _f32),
        pltpu.VMEM((NPAD,), _f32),
        pltpu.VMEM((SLW,), _f32),
        pltpu.VMEM((SLW,), _f32),
        pltpu.VMEM((NPAD2 // NS,), _f32),
        pltpu.VMEM((XCH, D), _f32),
        pltpu.VMEM((512 * 16,), _f32),
        pltpu.VMEM((D, 4), _f32),
        pltpu.VMEM((4, 4), _f32),
        pltpu.VMEM((4, 2), _f32),
        pltpu.VMEM((2, 4), _f32),
        pltpu.VMEM((4,), _f32),
        pltpu.VMEM((4,), _f32),
        pltpu.VMEM((2,), _f32),
        pltpu.VMEM((4,), _f32),
        pltpu.VMEM((16,), _f32),
    ],
)
def _sc_gcn(edge_hbm, x_hbm, w1_hbm, b1_hbm, w2_hbm, b2_hbm, w3_hbm, b3_hbm,
            wl_hbm, bl_hbm,
            out_hbm, h3_hbm,
            deg_sh, gs_sh, acc_sh, ea_v, eb_v, ia_v, ib_v, up_v, dinv_v,
            sl_v, pa_v, sh_v, xc_v, w1_v, w2d_v, wa_v, wb_v, wc_v,
            b1_v, b2_v, b3_v, bl_v, tb_v):
    sid = lax.axis_index("s")
    iota = lax.iota(_i32, 16)
    nodeoff = lax.shift_right_logical(iota, 2)
    feat = jnp.bitwise_and(iota, 3)
    blk = jnp.bitwise_and(iota, 12)
    ones16 = jnp.full((16,), 1.0, _f32)
    zeros16 = jnp.zeros((16,), _f32)
    fmask2 = jnp.where(feat < 2, 1.0, 0.0)
    feat01 = jnp.bitwise_and(feat, 1)
    nbase = sid * NPS

    pltpu.sync_copy(w1_hbm, w2d_v)
    pltpu.sync_copy(w2_hbm, wa_v)
    pltpu.sync_copy(w3_hbm, wb_v)
    pltpu.sync_copy(wl_hbm, wc_v)
    pltpu.sync_copy(b1_hbm, b1_v)
    pltpu.sync_copy(b2_hbm, b2_v)
    pltpu.sync_copy(b3_hbm, b3_v)
    pltpu.sync_copy(bl_hbm, bl_v)

    def wfill(kd, c):
        krow = jnp.broadcast_to(kd, (16,))
        for f in range(4):
            v = plsc.load_gather(w2d_v, [krow, jnp.broadcast_to(f, (16,))])
            w1_v[pl.ds(kd * 64 + f * 16, 16)] = _bf16r(v)
        return c

    lax.fori_loop(0, D, wfill, 0)

    def ofill(j, c):
        up_v[pl.ds(j * 16, 16)] = ones16
        return c

    lax.fori_loop(0, ECH // 16, ofill, 0)

    def zdfill(j, c):
        sl_v[pl.ds(j * 16, 16)] = zeros16
        return c

    lax.fori_loop(0, NPS // 16, zdfill, 0)
    pltpu.sync_copy(sl_v.at[pl.ds(0, NPS)], deg_sh.at[pl.ds(nbase, NPS)])
    plsc.subcore_barrier()

    def dbody(c, k):
        pltpu.sync_copy(edge_hbm.at[1, pl.ds(sid * EPT + c * ECH, ECH)], eb_v)
        pltpu.sync_copy(up_v.at[pl.ds(0, ECH)], deg_sh.at[eb_v], add=True)
        return k

    lax.fori_loop(0, NCH, dbody, 0)
    plsc.subcore_barrier()

    pltpu.sync_copy(deg_sh, dinv_v)

    def rbody(j, c):
        dinv_v[pl.ds(j * 16, 16)] = _rsqrt_newton(
            dinv_v[pl.ds(j * 16, 16)] + 1.0
        )
        return c

    lax.fori_loop(0, NPAD // 16, rbody, 0)

    def xchunk(cc, k):
        start = nbase + cc * XCH

        @pl.when(start < N)
        def _():
            pltpu.sync_copy(x_hbm.at[pl.ds(start, XCH), :], xc_v)

            def group(g, kk):
                rows16 = g * 16 + iota

                def kbody(kd, accs):
                    a0, a1, a2, a3 = accs
                    kcol = jnp.broadcast_to(kd, (16,))
                    xv = _bf16r(plsc.load_gather(xc_v, [rows16, kcol]))
                    wb = jnp.broadcast_to(kd * 64, (16,)) + iota
                    a0 = a0 + xv * plsc.load_gather(w1_v, [wb])
                    a1 = a1 + xv * plsc.load_gather(w1_v, [wb + 16])
                    a2 = a2 + xv * plsc.load_gather(w1_v, [wb + 32])
                    a3 = a3 + xv * plsc.load_gather(w1_v, [wb + 48])
                    return (a0, a1, a2, a3)

                accs = lax.fori_loop(
                    0, D, kbody, (zeros16, zeros16, zeros16, zeros16)
                )
                dix = plsc.load_gather(dinv_v, [start + rows16])
                lidx = (cc * XCH + rows16) * 4
                plsc.store_scatter(sl_v, [lidx], accs[0] * dix)
                plsc.store_scatter(sl_v, [lidx + 1], accs[1] * dix)
                plsc.store_scatter(sl_v, [lidx + 2], accs[2] * dix)
                plsc.store_scatter(sl_v, [lidx + 3], accs[3] * dix)
                return kk

            lax.fori_loop(0, XCH // 16, group, 0)

        @pl.when(start >= N)
        def _():
            def zg2(j, kk):
                sl_v[pl.ds(cc * XCH * 4 + j * 16, 16)] = zeros16
                return kk

            lax.fori_loop(0, XCH * 4 // 16, zg2, 0)

        return k

    lax.fori_loop(0, NXC, xchunk, 0)
    pltpu.sync_copy(sl_v, gs_sh.at[pl.ds(nbase * 4, SLW)])
    pltpu.sync_copy(sl_v, acc_sh.at[pl.ds(nbase * 4, SLW)])
    plsc.subcore_barrier()

    def edge_phase():
        def body(c, k):
            base = sid * EPT + c * ECH
            pltpu.sync_copy(edge_hbm.at[0, pl.ds(base, ECH)], ea_v)
            pltpu.sync_copy(edge_hbm.at[1, pl.ds(base, ECH)], eb_v)

            def ifill(j, kk):
                s4 = ea_v[pl.ds(j * 16, 16)] * 4
                d4 = eb_v[pl.ds(j * 16, 16)] * 4
                ia_v[pl.ds(j * 16, 16)] = s4
                ia_v[pl.ds(ECH + j * 16, 16)] = s4 + 1
                ia_v[pl.ds(2 * ECH + j * 16, 16)] = s4 + 2
                ia_v[pl.ds(3 * ECH + j * 16, 16)] = s4 + 3
                ib_v[pl.ds(j * 16, 16)] = d4
                ib_v[pl.ds(ECH + j * 16, 16)] = d4 + 1
                ib_v[pl.ds(2 * ECH + j * 16, 16)] = d4 + 2
                ib_v[pl.ds(3 * ECH + j * 16, 16)] = d4 + 3
                return kk

            lax.fori_loop(0, ECH // 16, ifill, 0)
            pltpu.sync_copy(gs_sh.at[ia_v], up_v)
            pltpu.sync_copy(up_v, acc_sh.at[ib_v], add=True)
            return k

        lax.fori_loop(0, NCH, body, 0)
        plsc.subcore_barrier()

    def epilogue(layer):
        pltpu.sync_copy(acc_sh.at[pl.ds(nbase * 4, SLW)], pa_v)
        if layer == 0:
            bvec = plsc.load_gather(b1_v, [feat])
            w0 = plsc.load_gather(wa_v, [jnp.broadcast_to(0, (16,)), feat])
            w1r = plsc.load_gather(wa_v, [jnp.broadcast_to(1, (16,)), feat])
            w2r = plsc.load_gather(wa_v, [jnp.broadcast_to(2, (16,)), feat])
            w3r = plsc.load_gather(wa_v, [jnp.broadcast_to(3, (16,)), feat])
        else:
            bvec = plsc.load_gather(b2_v, [feat])
            w0 = plsc.load_gather(wb_v, [jnp.broadcast_to(0, (16,)), feat01])
            w1r = plsc.load_gather(wb_v, [jnp.broadcast_to(1, (16,)), feat01])
            w2r = plsc.load_gather(wb_v, [jnp.broadcast_to(2, (16,)), feat01])
            w3r = plsc.load_gather(wb_v, [jnp.broadcast_to(3, (16,)), feat01])
            w0 = w0 * fmask2
            w1r = w1r * fmask2
            w2r = w2r * fmask2
            w3r = w3r * fmask2
        w0 = _bf16r(w0)
        w1r = _bf16r(w1r)
        w2r = _bf16r(w2r)
        w3r = _bf16r(w3r)

        def ebody(j, c):
            v = pa_v[pl.ds(j * 16, 16)]
            dix = plsc.load_gather(dinv_v, [nbase + j * 4 + nodeoff])
            v = v * dix + bvec
            t = _tanh16(v)
            tb_v[...] = _bf16r(t)
            acc = plsc.load_gather(tb_v, [blk]) * w0
            acc = acc + plsc.load_gather(tb_v, [blk + 1]) * w1r
            acc = acc + plsc.load_gather(tb_v, [blk + 2]) * w2r
            acc = acc + plsc.load_gather(tb_v, [blk + 3]) * w3r
            sl_v[pl.ds(j * 16, 16)] = acc * dix
            return c

        lax.fori_loop(0, SLW // 16, ebody, 0)
        pltpu.sync_copy(sl_v, acc_sh.at[pl.ds(nbase * 4, SLW)])
        pltpu.sync_copy(sl_v, gs_sh.at[pl.ds(nbase * 4, SLW)])
        plsc.subcore_barrier()

    edge_phase()
    epilogue(0)
    edge_phase()
    epilogue(1)
    edge_phase()

    pltpu.sync_copy(acc_sh.at[pl.ds(nbase * 4, SLW)], pa_v)
    b3vec = plsc.load_gather(b3_v, [feat01]) * fmask2
    blvec = plsc.load_gather(bl_v, [feat])
    wl0 = _bf16r(plsc.load_gather(wc_v, [jnp.broadcast_to(0, (16,)), feat]))
    wl1 = _bf16r(plsc.load_gather(wc_v, [jnp.broadcast_to(1, (16,)), feat]))

    def fbody(j, c):
        v = pa_v[pl.ds(j * 16, 16)]
        dix = plsc.load_gather(dinv_v, [nbase + j * 4 + nodeoff])
        h3 = v * dix + b3vec
        tb_v[...] = _bf16r(h3)
        acc = plsc.load_gather(tb_v, [blk]) * wl0
        acc = acc + plsc.load_gather(tb_v, [blk + 1]) * wl1
        sl_v[pl.ds(j * 16, 16)] = acc + blvec
        plsc.store_scatter(
            sh_v, [(j * 4 + nodeoff) * 2 + feat01], h3,
            mask=feat < 2,
        )
        return c

    lax.fori_loop(0, SLW // 16, fbody, 0)
    pltpu.sync_copy(sl_v, out_hbm.at[pl.ds(nbase * 4, SLW)])
    pltpu.sync_copy(sh_v, h3_hbm.at[pl.ds(nbase * 2, NPAD2 // NS)])


def kernel(x, edge_index, W1, b1, W2, b2, W3, b3, Wl, bl):
    out_full, h3_full = _sc_gcn(
        edge_index.astype(_i32), x, W1, b1, W2, b2, W3, b3, Wl, bl
    )
    return (out_full.reshape(NPAD, 4)[:N],
            h3_full.reshape(NPAD, 2)[:N])


# speedup vs baseline: 2.6420x; 1.0153x over previous
import functools

import jax
import jax.numpy as jnp
from jax import lax
from jax.experimental import pallas as pl
from jax.experimental.pallas import tpu as pltpu
from jax.experimental.pallas import tpu_sc as plsc

N = 10000
NPAD = 10240
D = 128
E = 320000
NS = 16
EPT = E // NS
ECH = 4000
NCH = EPT // ECH
ECH4 = ECH * 4
NPS = NPAD // NS
SLW = NPS * 4
NPAD4 = NPAD * 4
NPAD2 = NPAD * 2
XCH = 80
NXC = NPS // XCH

_f32 = jnp.float32
_i32 = jnp.int32


def _rsqrt_newton(d):
    d = jnp.maximum(d, 0.25)
    i = lax.bitcast_convert_type(d, _i32)
    i = 0x5F3759DF - lax.shift_right_logical(i, 1)
    y = lax.bitcast_convert_type(i, _f32)
    for _ in range(3):
        y = y * (1.5 - 0.5 * d * y * y)
    return y


def _bf16r(x):
    i = lax.bitcast_convert_type(x, _i32)
    r = i + 0x7FFF + jnp.bitwise_and(lax.shift_right_logical(i, 16), 1)
    r = jnp.bitwise_and(r, -65536)
    return lax.bitcast_convert_type(r, _f32)


def _tanh16(v):
    vm = jnp.minimum(jnp.maximum(v, -15.0), 15.0)
    e = jnp.exp(2.0 * vm)
    return 1.0 - 2.0 / (e + 1.0)


_sc_mesh = plsc.VectorSubcoreMesh(
    core_axis_name="c", subcore_axis_name="s", num_cores=1
)


@functools.partial(
    pl.kernel,
    out_type=(
        jax.ShapeDtypeStruct((NPAD4,), _f32),
        jax.ShapeDtypeStruct((NPAD2,), _f32),
    ),
    mesh=_sc_mesh,
    compiler_params=pltpu.CompilerParams(
        needs_layout_passes=False, use_tc_tiling_on_sc=False
    ),
    scratch_types=[
        pltpu.VMEM_SHARED((NPAD,), _f32),
        pltpu.VMEM_SHARED((NPAD4,), _f32),
        pltpu.VMEM_SHARED((NPAD4,), _f32),
        pltpu.VMEM((ECH,), _i32),
        pltpu.VMEM((ECH,), _i32),
        pltpu.VMEM((ECH4,), _i32),
        pltpu.VMEM((ECH4,), _i32),
        pltpu.VMEM((ECH4,), _f32),
        pltpu.VMEM((NPAD,), _f32),
        pltpu.VMEM((SLW,), _f32),
        pltpu.VMEM((SLW,), _f32),
        pltpu.VMEM((NPAD2 // NS,), _f32),
        pltpu.VMEM((XCH, D), _f32),
        pltpu.VMEM((512 * 16,), _f32),
        pltpu.VMEM((D, 4), _f32),
        pltpu.VMEM((4, 4), _f32),
        pltpu.VMEM((4, 2), _f32),
        pltpu.VMEM((2, 4), _f32),
        pltpu.VMEM((4,), _f32),
        pltpu.VMEM((4,), _f32),
        pltpu.VMEM((2,), _f32),
        pltpu.VMEM((4,), _f32),
        pltpu.VMEM((16,), _f32),
    ],
)
def _sc_gcn(edge_hbm, x_hbm, w1_hbm, b1_hbm, w2_hbm, b2_hbm, w3_hbm, b3_hbm,
            wl_hbm, bl_hbm,
            out_hbm, h3_hbm,
            deg_sh, gs_sh, acc_sh, ea_v, eb_v, ia_v, ib_v, up_v, dinv_v,
            sl_v, pa_v, sh_v, xc_v, w1_v, w2d_v, wa_v, wb_v, wc_v,
            b1_v, b2_v, b3_v, bl_v, tb_v):
    sid = lax.axis_index("s")
    iota = lax.iota(_i32, 16)
    nodeoff = lax.shift_right_logical(iota, 2)
    feat = jnp.bitwise_and(iota, 3)
    blk = jnp.bitwise_and(iota, 12)
    ones16 = jnp.full((16,), 1.0, _f32)
    zeros16 = jnp.zeros((16,), _f32)
    fmask2 = jnp.where(feat < 2, 1.0, 0.0)
    feat01 = jnp.bitwise_and(feat, 1)
    nbase = sid * NPS

    pltpu.sync_copy(w1_hbm, w2d_v)
    pltpu.sync_copy(w2_hbm, wa_v)
    pltpu.sync_copy(w3_hbm, wb_v)
    pltpu.sync_copy(wl_hbm, wc_v)
    pltpu.sync_copy(b1_hbm, b1_v)
    pltpu.sync_copy(b2_hbm, b2_v)
    pltpu.sync_copy(b3_hbm, b3_v)
    pltpu.sync_copy(bl_hbm, bl_v)

    def wfill(kd, c):
        krow = jnp.broadcast_to(kd, (16,))
        for f in range(4):
            v = plsc.load_gather(w2d_v, [krow, jnp.broadcast_to(f, (16,))])
            w1_v[pl.ds(kd * 64 + f * 16, 16)] = _bf16r(v)
        return c

    lax.fori_loop(0, D, wfill, 0)

    def ofill(j, c):
        up_v[pl.ds(j * 16, 16)] = ones16
        return c

    lax.fori_loop(0, ECH // 16, ofill, 0)

    def zdfill(j, c):
        sl_v[pl.ds(j * 16, 16)] = zeros16
        return c

    lax.fori_loop(0, NPS // 16, zdfill, 0)
    pltpu.sync_copy(sl_v.at[pl.ds(0, NPS)], deg_sh.at[pl.ds(nbase, NPS)])
    plsc.subcore_barrier()

    def dbody(c, k):
        pltpu.sync_copy(edge_hbm.at[1, pl.ds(sid * EPT + c * ECH, ECH)], eb_v)
        pltpu.sync_copy(up_v.at[pl.ds(0, ECH)], deg_sh.at[eb_v], add=True)
        return k

    lax.fori_loop(0, NCH, dbody, 0)
    plsc.subcore_barrier()

    pltpu.sync_copy(deg_sh, dinv_v)

    def rbody(j, c):
        dinv_v[pl.ds(j * 16, 16)] = _rsqrt_newton(
            dinv_v[pl.ds(j * 16, 16)] + 1.0
        )
        return c

    lax.fori_loop(0, NPAD // 16, rbody, 0)

    def xchunk(cc, k):
        start = nbase + cc * XCH

        @pl.when(start < N)
        def _():
            pltpu.sync_copy(x_hbm.at[pl.ds(start, XCH), :], xc_v)
            ngr = XCH // 16

            def kbody(kd, accs):
                kcol = jnp.broadcast_to(kd, (16,))
                wb = jnp.broadcast_to(kd * 64, (16,)) + iota
                w0 = plsc.load_gather(w1_v, [wb])
                w1w = plsc.load_gather(w1_v, [wb + 16])
                w2w = plsc.load_gather(w1_v, [wb + 32])
                w3w = plsc.load_gather(w1_v, [wb + 48])
                out = []
                for g in range(ngr):
                    rows16 = g * 16 + iota
                    xv = _bf16r(plsc.load_gather(xc_v, [rows16, kcol]))
                    out.append(accs[g * 4 + 0] + xv * w0)
                    out.append(accs[g * 4 + 1] + xv * w1w)
                    out.append(accs[g * 4 + 2] + xv * w2w)
                    out.append(accs[g * 4 + 3] + xv * w3w)
                return tuple(out)

            accs = lax.fori_loop(0, D, kbody, (zeros16,) * (4 * ngr))
            for g in range(ngr):
                rows16 = g * 16 + iota
                dix = plsc.load_gather(dinv_v, [start + rows16])
                lidx = (cc * XCH + rows16) * 4
                plsc.store_scatter(sl_v, [lidx], accs[g * 4 + 0] * dix)
                plsc.store_scatter(sl_v, [lidx + 1], accs[g * 4 + 1] * dix)
                plsc.store_scatter(sl_v, [lidx + 2], accs[g * 4 + 2] * dix)
                plsc.store_scatter(sl_v, [lidx + 3], accs[g * 4 + 3] * dix)

        @pl.when(start >= N)
        def _():
            def zg2(j, kk):
                sl_v[pl.ds(cc * XCH * 4 + j * 16, 16)] = zeros16
                return kk

            lax.fori_loop(0, XCH * 4 // 16, zg2, 0)

        return k

    lax.fori_loop(0, NXC, xchunk, 0)
    pltpu.sync_copy(sl_v, gs_sh.at[pl.ds(nbase * 4, SLW)])
    pltpu.sync_copy(sl_v, acc_sh.at[pl.ds(nbase * 4, SLW)])
    plsc.subcore_barrier()

    def edge_phase():
        def body(c, k):
            base = sid * EPT + c * ECH
            pltpu.sync_copy(edge_hbm.at[0, pl.ds(base, ECH)], ea_v)
            pltpu.sync_copy(edge_hbm.at[1, pl.ds(base, ECH)], eb_v)

            def ifill(j, kk):
                s4 = ea_v[pl.ds(j * 16, 16)] * 4
                d4 = eb_v[pl.ds(j * 16, 16)] * 4
                ia_v[pl.ds(j * 16, 16)] = s4
                ia_v[pl.ds(ECH + j * 16, 16)] = s4 + 1
                ia_v[pl.ds(2 * ECH + j * 16, 16)] = s4 + 2
                ia_v[pl.ds(3 * ECH + j * 16, 16)] = s4 + 3
                ib_v[pl.ds(j * 16, 16)] = d4
                ib_v[pl.ds(ECH + j * 16, 16)] = d4 + 1
                ib_v[pl.ds(2 * ECH + j * 16, 16)] = d4 + 2
                ib_v[pl.ds(3 * ECH + j * 16, 16)] = d4 + 3
                return kk

            lax.fori_loop(0, ECH // 16, ifill, 0)
            pltpu.sync_copy(gs_sh.at[ia_v], up_v)
            pltpu.sync_copy(up_v, acc_sh.at[ib_v], add=True)
            return k

        lax.fori_loop(0, NCH, body, 0)
        plsc.subcore_barrier()

    def epilogue(layer):
        pltpu.sync_copy(acc_sh.at[pl.ds(nbase * 4, SLW)], pa_v)
        if layer == 0:
            bvec = plsc.load_gather(b1_v, [feat])
            w0 = plsc.load_gather(wa_v, [jnp.broadcast_to(0, (16,)), feat])
            w1r = plsc.load_gather(wa_v, [jnp.broadcast_to(1, (16,)), feat])
            w2r = plsc.load_gather(wa_v, [jnp.broadcast_to(2, (16,)), feat])
            w3r = plsc.load_gather(wa_v, [jnp.broadcast_to(3, (16,)), feat])
        else:
            bvec = plsc.load_gather(b2_v, [feat])
            w0 = plsc.load_gather(wb_v, [jnp.broadcast_to(0, (16,)), feat01])
            w1r = plsc.load_gather(wb_v, [jnp.broadcast_to(1, (16,)), feat01])
            w2r = plsc.load_gather(wb_v, [jnp.broadcast_to(2, (16,)), feat01])
            w3r = plsc.load_gather(wb_v, [jnp.broadcast_to(3, (16,)), feat01])
            w0 = w0 * fmask2
            w1r = w1r * fmask2
            w2r = w2r * fmask2
            w3r = w3r * fmask2
        w0 = _bf16r(w0)
        w1r = _bf16r(w1r)
        w2r = _bf16r(w2r)
        w3r = _bf16r(w3r)

        def ebody(j, c):
            v = pa_v[pl.ds(j * 16, 16)]
            dix = plsc.load_gather(dinv_v, [nbase + j * 4 + nodeoff])
            v = v * dix + bvec
            t = _tanh16(v)
            tb_v[...] = _bf16r(t)
            acc = plsc.load_gather(tb_v, [blk]) * w0
            acc = acc + plsc.load_gather(tb_v, [blk + 1]) * w1r
            acc = acc + plsc.load_gather(tb_v, [blk + 2]) * w2r
            acc = acc + plsc.load_gather(tb_v, [blk + 3]) * w3r
            sl_v[pl.ds(j * 16, 16)] = acc * dix
            return c

        lax.fori_loop(0, SLW // 16, ebody, 0)
        pltpu.sync_copy(sl_v, acc_sh.at[pl.ds(nbase * 4, SLW)])
        pltpu.sync_copy(sl_v, gs_sh.at[pl.ds(nbase * 4, SLW)])
        plsc.subcore_barrier()

    edge_phase()
    epilogue(0)
    edge_phase()
    epilogue(1)
    edge_phase()

    pltpu.sync_copy(acc_sh.at[pl.ds(nbase * 4, SLW)], pa_v)
    b3vec = plsc.load_gather(b3_v, [feat01]) * fmask2
    blvec = plsc.load_gather(bl_v, [feat])
    wl0 = _bf16r(plsc.load_gather(wc_v, [jnp.broadcast_to(0, (16,)), feat]))
    wl1 = _bf16r(plsc.load_gather(wc_v, [jnp.broadcast_to(1, (16,)), feat]))

    def fbody(j, c):
        v = pa_v[pl.ds(j * 16, 16)]
        dix = plsc.load_gather(dinv_v, [nbase + j * 4 + nodeoff])
        h3 = v * dix + b3vec
        tb_v[...] = _bf16r(h3)
        acc = plsc.load_gather(tb_v, [blk]) * wl0
        acc = acc + plsc.load_gather(tb_v, [blk + 1]) * wl1
        sl_v[pl.ds(j * 16, 16)] = acc + blvec
        plsc.store_scatter(
            sh_v, [(j * 4 + nodeoff) * 2 + feat01], h3,
            mask=feat < 2,
        )
        return c

    lax.fori_loop(0, SLW // 16, fbody, 0)
    pltpu.sync_copy(sl_v, out_hbm.at[pl.ds(nbase * 4, SLW)])
    pltpu.sync_copy(sh_v, h3_hbm.at[pl.ds(nbase * 2, NPAD2 // NS)])


def kernel(x, edge_index, W1, b1, W2, b2, W3, b3, Wl, bl):
    out_full, h3_full = _sc_gcn(
        edge_index.astype(_i32), x, W1, b1, W2, b2, W3, b3, Wl, bl
    )
    return (out_full.reshape(NPAD, 4)[:N],
            h3_full.reshape(NPAD, 2)[:N])
